# baseline (speedup 1.0000x reference)
import functools

import jax
import jax.numpy as jnp
from jax import lax
from jax.experimental import pallas as pl
from jax.experimental.pallas import tpu as pltpu
from jax.experimental.pallas import tpu_sc as plsc

B = 16384
D = 128
NC = 2
NS = 16
NW = NC * NS
BPW = B // NW
CH = 128
NCH = BPW // CH
NBUF = 3
L = 16
NJ = D // L


def _sc_body(ui_hbm, ii_hbm, ut_hbm, it_hbm, w_hbm, b_hbm, out_hbm,
             ui_v, ii_v, u0, i0, u1, i1, u2, i2, w_v, b_v, out_v,
             sem0, sem1, sem2):
    c_id = lax.axis_index("c")
    s_id = lax.axis_index("s")
    wid = s_id * NC + c_id

    c1 = pltpu.async_copy(ui_hbm.at[pl.ds(wid * NCH, NCH)], ui_v, sem0)
    c2 = pltpu.async_copy(ii_hbm.at[pl.ds(wid * NCH, NCH)], ii_v, sem0)
    c3 = pltpu.async_copy(w_hbm, w_v, sem0)
    c4 = pltpu.async_copy(b_hbm, b_v, sem0)
    c1.wait()
    c2.wait()
    c3.wait()
    c4.wait()

    w_regs = [w_v[pl.ds(j * L, L)] for j in range(NJ)]
    b_reg = b_v[...]
    lane = lax.iota(jnp.int32, L)
    halfidx = ((lane & 1) << 3) | ((lane & 2) << 1) | ((lane & 4) >> 1)
    perm_idx = {s: lane ^ s for s in (8, 4, 2, 1)}
    stage_mask = {s: (lane & s) == 0 for s in (8, 4, 2)}
    low_half = lane < 8

    gather_dn = lax.GatherDimensionNumbers(
        offset_dims=(), collapsed_slice_dims=(0,), start_index_map=(0,))

    def lane_perm(v, idx):
        return lax.gather(v, idx[:, None], gather_dn, (1,),
                          mode=lax.GatherScatterMode.PROMISE_IN_BOUNDS)

    def merge(a, b, s):
        return jnp.where(stage_mask[s],
                         a + lane_perm(a, perm_idx[s]),
                         lane_perm(b, perm_idx[s]) + b)

    bufs = [(u0, i0, sem0), (u1, i1, sem1), (u2, i2, sem2)]
    pend = [None] * NBUF

    def fire(c):
        u_rows, i_rows, sem = bufs[c % NBUF]
        d1 = pltpu.async_copy(ut_hbm.at[ui_v.at[c]], u_rows, sem)
        d2 = pltpu.async_copy(it_hbm.at[ii_v.at[c]], i_rows, sem)
        pend[c % NBUF] = (d1, d2)

    fire(0)
    fire(1)
    for c in range(NCH):
        if c + 2 < NCH:
            fire(c + 2)
        d1, d2 = pend[c % NBUF]
        d1.wait()
        d2.wait()
        u_rows, i_rows, _ = bufs[c % NBUF]

        @plsc.parallel_loop(0, CH // L)
        def group_body(g, u_rows=u_rows, i_rows=i_rows, c=c):
            def row_acc(row):
                even = odd = None
                for j in range(NJ):
                    u_j = u_rows[row, pl.ds(j * L, L)]
                    i_j = i_rows[row, pl.ds(j * L, L)]
                    t = (u_j * i_j) * w_regs[j]
                    if j % 2 == 0:
                        even = t if even is None else even + t
                    else:
                        odd = t if odd is None else odd + t
                return even + odd

            def tree(lo, n):
                if n == 1:
                    return row_acc(g * L + lo)
                a = tree(lo, n // 2)
                b = tree(lo + n // 2, n // 2)
                return merge(a, b, 16 // n)

            def half(lo):
                z = tree(lo, 8)
                return z + lane_perm(z, perm_idx[1])

            z_a = half(0)
            z_b = half(8)
            z = jnp.where(low_half,
                          lane_perm(z_a, halfidx),
                          lane_perm(z_b, halfidx)) + b_reg
            out_v[pl.ds(c * CH + g * L, L)] = 1.0 / (1.0 + jnp.exp(-z))

    pltpu.sync_copy(out_v, out_hbm.at[pl.ds(wid * BPW, BPW)])


@functools.partial(
    pl.kernel,
    out_type=jax.ShapeDtypeStruct((B,), jnp.float32),
    mesh=plsc.VectorSubcoreMesh(core_axis_name="c", subcore_axis_name="s"),
    scratch_types=[
        pltpu.VMEM((NCH, CH), jnp.int32),
        pltpu.VMEM((NCH, CH), jnp.int32),
        pltpu.VMEM((CH, D), jnp.float32),
        pltpu.VMEM((CH, D), jnp.float32),
        pltpu.VMEM((CH, D), jnp.float32),
        pltpu.VMEM((CH, D), jnp.float32),
        pltpu.VMEM((CH, D), jnp.float32),
        pltpu.VMEM((CH, D), jnp.float32),
        pltpu.VMEM((D,), jnp.float32),
        pltpu.VMEM((L,), jnp.float32),
        pltpu.VMEM((BPW,), jnp.float32),
        pltpu.SemaphoreType.DMA,
        pltpu.SemaphoreType.DMA,
        pltpu.SemaphoreType.DMA,
    ],
)
def _gimf_sc(ui_hbm, ii_hbm, ut_hbm, it_hbm, w_hbm, b_hbm, out_hbm,
             ui_v, ii_v, u0, i0, u1, i1, u2, i2, w_v, b_v, out_v,
             sem0, sem1, sem2):
    _sc_body(ui_hbm, ii_hbm, ut_hbm, it_hbm, w_hbm, b_hbm, out_hbm,
             ui_v, ii_v, u0, i0, u1, i1, u2, i2, w_v, b_v, out_v,
             sem0, sem1, sem2)


def kernel(user_indices, item_indices, user_table, item_table, W, b):
    ui = user_indices.astype(jnp.int32).reshape(B // CH, CH)
    ii = item_indices.astype(jnp.int32).reshape(B // CH, CH)
    w_flat = W.reshape(D).astype(jnp.float32)
    b_vec = jnp.broadcast_to(b.reshape(()), (L,)).astype(jnp.float32)
    out = _gimf_sc(ui, ii, user_table, item_table, w_flat, b_vec)
    return out.reshape(B, 1)


# speedup vs baseline: 1.0812x; 1.0812x over previous
import functools

import jax
import jax.numpy as jnp
from jax import lax
from jax.experimental import pallas as pl
from jax.experimental.pallas import tpu as pltpu
from jax.experimental.pallas import tpu_sc as plsc

B = 16384
D = 128
NC = 2
NS = 16
NW = NC * NS
BPW = B // NW
CH = 128
NCH = BPW // CH
L = 16
NJ = D // L

_BITREV = [0, 8, 4, 12, 2, 10, 6, 14, 1, 9, 5, 13, 3, 11, 7, 15]


def _sc_body(ui_hbm, ii_hbm, ut_hbm, it_hbm, w_hbm, b_hbm, out_hbm,
             ui_v, ii_v, u0, i0, u1, i1, w_v, b_v, out_v, sem0, sem1):
    c_id = lax.axis_index("c")
    s_id = lax.axis_index("s")
    wid = s_id * NC + c_id

    c1 = pltpu.async_copy(ui_hbm.at[pl.ds(wid * NCH, NCH)], ui_v, sem1)
    c2 = pltpu.async_copy(ii_hbm.at[pl.ds(wid * NCH, NCH)], ii_v, sem1)
    c3 = pltpu.async_copy(w_hbm, w_v, sem1)
    c4 = pltpu.async_copy(b_hbm, b_v, sem1)
    c1.wait()
    c2.wait()
    lane = lax.iota(jnp.int32, L)
    scat = (((lane & 1) << 3) | ((lane & 2) << 1)
            | ((lane & 4) >> 1) | ((lane & 8) >> 3))
    perm_idx = {s: lane ^ s for s in (8, 4, 2, 1)}
    stage_mask = {s: (lane & s) == 0 for s in (8, 4, 2, 1)}

    gather_dn = lax.GatherDimensionNumbers(
        offset_dims=(), collapsed_slice_dims=(0,), start_index_map=(0,))

    def lane_perm(v, idx):
        return lax.gather(v, idx[:, None], gather_dn, (1,),
                          mode=lax.GatherScatterMode.PROMISE_IN_BOUNDS)

    def merge(a, b, s):
        return jnp.where(stage_mask[s],
                         a + lane_perm(a, perm_idx[s]),
                         lane_perm(b, perm_idx[s]) + b)

    bufs = [(u0, i0, sem0), (u1, i1, sem1)]
    pend = [None, None]

    def fire(c):
        u_rows, i_rows, sem = bufs[c % 2]
        d1 = pltpu.async_copy(ut_hbm.at[ui_v.at[c]], u_rows, sem)
        d2 = pltpu.async_copy(it_hbm.at[ii_v.at[c]], i_rows, sem)
        pend[c % 2] = (d1, d2)

    fire(0)
    c3.wait()
    c4.wait()
    w_regs = [w_v[pl.ds(j * L, L)] for j in range(NJ)]
    b_reg = b_v[...]

    for c in range(NCH):
        if c + 1 < NCH:
            fire(c + 1)
        d1, d2 = pend[c % 2]
        d1.wait()
        d2.wait()
        u_rows, i_rows, _ = bufs[c % 2]

        @plsc.parallel_loop(0, CH // L)
        def group_body(g, u_rows=u_rows, i_rows=i_rows, c=c):
            def row_acc(row):
                even = odd = None
                for j in range(NJ):
                    u_j = u_rows[row, pl.ds(j * L, L)]
                    i_j = i_rows[row, pl.ds(j * L, L)]
                    t = (u_j * i_j) * w_regs[j]
                    if j % 2 == 0:
                        even = t if even is None else even + t
                    else:
                        odd = t if odd is None else odd + t
                return even + odd

            def tree(lo, n):
                if n == 1:
                    return row_acc(g * L + lo)
                a = tree(lo, n // 2)
                b = tree(lo + n // 2, n // 2)
                return merge(a, b, 16 // n)

            z = tree(0, L) + b_reg
            res = 1.0 / (1.0 + jnp.exp(-z))
            out_v[pl.ds(c * CH + g * L, L)] = lane_perm(res, scat)

    pltpu.sync_copy(out_v, out_hbm.at[pl.ds(wid * BPW, BPW)])


@functools.partial(
    pl.kernel,
    out_type=jax.ShapeDtypeStruct((B,), jnp.float32),
    mesh=plsc.VectorSubcoreMesh(core_axis_name="c", subcore_axis_name="s"),
    scratch_types=[
        pltpu.VMEM((NCH, CH), jnp.int32),
        pltpu.VMEM((NCH, CH), jnp.int32),
        pltpu.VMEM((CH, D), jnp.float32),
        pltpu.VMEM((CH, D), jnp.float32),
        pltpu.VMEM((CH, D), jnp.float32),
        pltpu.VMEM((CH, D), jnp.float32),
        pltpu.VMEM((D,), jnp.float32),
        pltpu.VMEM((L,), jnp.float32),
        pltpu.VMEM((BPW,), jnp.float32),
        pltpu.SemaphoreType.DMA,
        pltpu.SemaphoreType.DMA,
    ],
)
def _gimf_sc(ui_hbm, ii_hbm, ut_hbm, it_hbm, w_hbm, b_hbm, out_hbm,
             ui_v, ii_v, u0, i0, u1, i1, w_v, b_v, out_v, sem0, sem1):
    _sc_body(ui_hbm, ii_hbm, ut_hbm, it_hbm, w_hbm, b_hbm, out_hbm,
             ui_v, ii_v, u0, i0, u1, i1, w_v, b_v, out_v, sem0, sem1)


def kernel(user_indices, item_indices, user_table, item_table, W, b):
    ui = user_indices.astype(jnp.int32).reshape(B // CH, CH)
    ii = item_indices.astype(jnp.int32).reshape(B // CH, CH)
    w_flat = W.reshape(D).astype(jnp.float32)
    b_vec = jnp.broadcast_to(b.reshape(()), (L,)).astype(jnp.float32)
    out = _gimf_sc(ui, ii, user_table, item_table, w_flat, b_vec)
    return out.reshape(B, 1)
